# baseline (speedup 1.0000x reference)
import functools

import jax
import jax.numpy as jnp
from jax import lax
from jax.experimental import pallas as pl
from jax.experimental.pallas import tpu as pltpu
from jax.experimental.pallas import tpu_sc as plsc

N = 10000
D_IN = 128
D_H1 = 128
D_H2 = 64

NUM_TILES = 32
SUBCORES = 16
BATCH = 128
ACC_ROWS = 10112
ROWS_PER_TILE = ACC_ROWS // SUBCORES
ROW_BLK = 1000

S_C0 = 40
S_C1 = 40


def _sc_degree(dst3, zeros_init, e0_rows):
  streams = dst3.shape[1]
  mesh = plsc.VectorSubcoreMesh(core_axis_name="c", subcore_axis_name="s")

  @functools.partial(
      pl.kernel,
      mesh=mesh,
      out_type=jax.ShapeDtypeStruct((2, ACC_ROWS, D_H1), jnp.float32),
      scratch_types=[
          pltpu.VMEM((streams, BATCH), jnp.int32),
          pltpu.VMEM((BATCH, D_H1), jnp.float32),
          pltpu.VMEM_SHARED((ACC_ROWS, D_H1), jnp.float32),
      ],
  )
  def k(dst_hbm, z_hbm, e0_hbm, out_hbm, dst_v, ones_v, acc):
    c = lax.axis_index("c")
    s = lax.axis_index("s")
    wid = s * 2 + c
    pltpu.sync_copy(z_hbm, acc.at[pl.ds(s * ROWS_PER_TILE, ROWS_PER_TILE)])
    pltpu.sync_copy(e0_hbm, ones_v)
    pltpu.sync_copy(dst_hbm.at[wid], dst_v)
    plsc.subcore_barrier()

    def body(j, carry):
      pltpu.sync_copy(ones_v, acc.at[dst_v.at[j]], add=True)
      return carry

    lax.fori_loop(0, streams, body, 0)
    plsc.subcore_barrier()
    pltpu.sync_copy(
        acc.at[pl.ds(s * ROWS_PER_TILE, ROWS_PER_TILE)],
        out_hbm.at[c, pl.ds(s * ROWS_PER_TILE, ROWS_PER_TILE)])

  return k(dst3, zeros_init, e0_rows)


def _sc_aggregate(y, src2, dst2, zeros_init):
  mesh = plsc.VectorSubcoreMesh(core_axis_name="c", subcore_axis_name="s")

  NBUF = 2
  SMAX = max(S_C0, S_C1)

  @functools.partial(
      pl.kernel,
      mesh=mesh,
      out_type=jax.ShapeDtypeStruct((2, ACC_ROWS, D_H1), jnp.float32),
      scratch_types=[
          pltpu.VMEM((SMAX, BATCH), jnp.int32),
          pltpu.VMEM((SMAX, BATCH), jnp.int32),
          [pltpu.VMEM((BATCH, D_H1), jnp.float32)] * NBUF,
          pltpu.VMEM_SHARED((ACC_ROWS, D_H1), jnp.float32),
          [pltpu.SemaphoreType.DMA] * NBUF,
          [pltpu.SemaphoreType.DMA] * NBUF,
      ],
  )
  def k(y_hbm, src_hbm, dst_hbm, z_hbm, out_hbm, src_v, dst_v, rows, acc,
        gsem, ssem):
    c = lax.axis_index("c")
    s = lax.axis_index("s")
    pltpu.sync_copy(z_hbm, acc.at[pl.ds(s * ROWS_PER_TILE, ROWS_PER_TILE)])
    plsc.subcore_barrier()

    def run(ns, base, yref):
      pltpu.sync_copy(src_hbm.at[pl.ds(base, ns)], src_v.at[pl.ds(0, ns)])
      pltpu.sync_copy(dst_hbm.at[pl.ds(base, ns)], dst_v.at[pl.ds(0, ns)])
      gd = [None] * NBUF
      sd = [None] * NBUF
      for b in range(min(NBUF, ns)):
        gd[b] = pltpu.async_copy(yref.at[src_v.at[b]], rows[b], gsem[b])
      for j in range(ns):
        p = j % NBUF
        gd[p].wait()
        sd[p] = pltpu.async_copy(rows[p], acc.at[dst_v.at[j]], ssem[p],
                                 add=True)
        nj = j + NBUF
        sd[p].wait()
        if nj < ns:
          gd[p] = pltpu.async_copy(yref.at[src_v.at[nj]], rows[p], gsem[p])

    @pl.when(c == 0)
    def _():
      run(S_C0, s * S_C0, y_hbm)

    @pl.when(c == 1)
    def _():
      run(S_C1, SUBCORES * S_C0 + s * S_C1, y_hbm)

    plsc.subcore_barrier()
    pltpu.sync_copy(
        acc.at[pl.ds(s * ROWS_PER_TILE, ROWS_PER_TILE)],
        out_hbm.at[c, pl.ds(s * ROWS_PER_TILE, ROWS_PER_TILE)])

  return k(y, src2, dst2, zeros_init)


def _tc_xw_scale(X, W1, degs):
  grid = N // ROW_BLK

  def body(x_ref, w_ref, deg_ref, y_ref, dinv_ref):
    deg = deg_ref[0, :, 0] + deg_ref[1, :, 0] + 1.0
    dinv = lax.rsqrt(jnp.maximum(deg, 1.0))
    xw = jnp.dot(x_ref[...], w_ref[...], preferred_element_type=jnp.float32)
    y_ref[...] = xw * dinv[:, None]
    dinv_ref[...] = dinv[:, None]

  return pl.pallas_call(
      body,
      grid=(grid,),
      in_specs=[
          pl.BlockSpec((ROW_BLK, D_IN), lambda i: (i, 0)),
          pl.BlockSpec((D_IN, D_H1), lambda i: (0, 0)),
          pl.BlockSpec((2, ROW_BLK, D_H1), lambda i: (0, i, 0)),
      ],
      out_specs=[
          pl.BlockSpec((ROW_BLK, D_H1), lambda i: (i, 0)),
          pl.BlockSpec((ROW_BLK, 1), lambda i: (i, 0)),
      ],
      out_shape=[
          jax.ShapeDtypeStruct((N, D_H1), jnp.float32),
          jax.ShapeDtypeStruct((N, 1), jnp.float32),
      ],
  )(X, W1, degs)


def _tc_combine(s_parts, y, dinv, b):
  grid = N // ROW_BLK

  def body(s_ref, y_ref, dinv_ref, b_ref, out_ref):
    di = dinv_ref[...]
    t = (s_ref[0] + s_ref[1] + y_ref[...]) * di + b_ref[...]
    out_ref[...] = jnp.maximum(t, 0.0) * di

  return pl.pallas_call(
      body,
      grid=(grid,),
      in_specs=[
          pl.BlockSpec((2, ROW_BLK, D_H1), lambda i: (0, i, 0)),
          pl.BlockSpec((ROW_BLK, D_H1), lambda i: (i, 0)),
          pl.BlockSpec((ROW_BLK, 1), lambda i: (i, 0)),
          pl.BlockSpec((1, D_H1), lambda i: (0, 0)),
      ],
      out_specs=pl.BlockSpec((ROW_BLK, D_H1), lambda i: (i, 0)),
      out_shape=jax.ShapeDtypeStruct((N, D_H1), jnp.float32),
  )(s_parts, y, dinv, b)


def _tc_heads(s_parts, y, dinv, Wm, bm, Wl, bl, noise):
  grid = N // ROW_BLK

  def body(s_ref, y_ref, dinv_ref, wm_ref, bm_ref, wl_ref, bl_ref, n_ref,
           z_ref):
    g = (s_ref[0] + s_ref[1] + y_ref[...]) * dinv_ref[...]
    mean = jnp.maximum(
        jnp.dot(g, wm_ref[...], preferred_element_type=jnp.float32)
        + bm_ref[...], 0.0)
    logstd = jnp.maximum(
        jnp.dot(g, wl_ref[...], preferred_element_type=jnp.float32)
        + bl_ref[...], 0.0)
    z_ref[...] = n_ref[...] * jnp.exp(logstd) + mean

  return pl.pallas_call(
      body,
      grid=(grid,),
      in_specs=[
          pl.BlockSpec((2, ROW_BLK, D_H1), lambda i: (0, i, 0)),
          pl.BlockSpec((ROW_BLK, D_H1), lambda i: (i, 0)),
          pl.BlockSpec((ROW_BLK, 1), lambda i: (i, 0)),
          pl.BlockSpec((D_H1, D_H2), lambda i: (0, 0)),
          pl.BlockSpec((1, D_H2), lambda i: (0, 0)),
          pl.BlockSpec((D_H1, D_H2), lambda i: (0, 0)),
          pl.BlockSpec((1, D_H2), lambda i: (0, 0)),
          pl.BlockSpec((ROW_BLK, D_H2), lambda i: (i, 0)),
      ],
      out_specs=pl.BlockSpec((ROW_BLK, D_H2), lambda i: (i, 0)),
      out_shape=jax.ShapeDtypeStruct((N, D_H2), jnp.float32),
  )(s_parts, y, dinv, Wm, bm, Wl, bl, noise)


def _tc_decode(Z):
  DEC_I = 1024
  DEC_J = 4096
  gi = pl.cdiv(N, DEC_I)
  gj = pl.cdiv(N, DEC_J)

  def body(a_ref, b_ref, out_ref):
    logits = lax.dot_general(
        a_ref[...], b_ref[...], (((1,), (1,)), ((), ())),
        preferred_element_type=jnp.float32)
    out_ref[...] = jax.nn.sigmoid(logits)

  return pl.pallas_call(
      body,
      grid=(gi, gj),
      in_specs=[
          pl.BlockSpec((DEC_I, D_H2), lambda i, j: (i, 0)),
          pl.BlockSpec((DEC_J, D_H2), lambda i, j: (j, 0)),
      ],
      out_specs=pl.BlockSpec((DEC_I, DEC_J), lambda i, j: (i, j)),
      out_shape=jax.ShapeDtypeStruct((N, N), jnp.float32),
  )(Z, Z)


def kernel(X, adj, W1, b1, Wm, bm, Wl, bl):
  E = adj.shape[1]
  src = adj[0].astype(jnp.int32)
  dst = adj[1].astype(jnp.int32)

  chunk = NUM_TILES * BATCH
  e_pad_deg = ((E + chunk - 1) // chunk) * chunk
  e_pad_agg = SUBCORES * (S_C0 + S_C1) * BATCH

  def padded(vec, n_to, dummy):
    return jnp.concatenate([vec, dummy[:n_to - E]])

  dummy_src = jnp.arange(max(e_pad_deg, e_pad_agg) - E, dtype=jnp.int32) % N
  dummy_dst = N + jnp.arange(max(e_pad_deg, e_pad_agg) - E,
                             dtype=jnp.int32) % (ACC_ROWS - N)

  streams = e_pad_deg // chunk
  dst3 = padded(dst, e_pad_deg, dummy_dst).reshape(NUM_TILES, streams, BATCH)
  src2 = padded(src, e_pad_agg, dummy_src).reshape(-1, BATCH)
  dst2 = padded(dst, e_pad_agg, dummy_dst).reshape(-1, BATCH)

  zeros_feat = jnp.zeros((ROWS_PER_TILE, D_H1), jnp.float32)
  zeros_deg = zeros_feat
  e0_rows = jnp.zeros((BATCH, D_H1), jnp.float32).at[:, 0].set(1.0)

  degs = _sc_degree(dst3, zeros_deg, e0_rows)
  y1, dinv = _tc_xw_scale(X, W1, degs)
  s1 = _sc_aggregate(y1, src2, dst2, zeros_feat)
  y2 = _tc_combine(s1, y1, dinv, b1.reshape(1, D_H1))
  s2 = _sc_aggregate(y2, src2, dst2, zeros_feat)
  noise = jax.random.normal(jax.random.key(42), (N, D_H2), dtype=jnp.float32)
  Z = _tc_heads(s2, y2, dinv, Wm, bm.reshape(1, D_H2), Wl,
                bl.reshape(1, D_H2), noise)
  A_pred = _tc_decode(Z)
  return (Z, A_pred)


# speedup vs baseline: 1.0712x; 1.0712x over previous
import functools

import jax
import jax.numpy as jnp
from jax import lax
from jax.experimental import pallas as pl
from jax.experimental.pallas import tpu as pltpu
from jax.experimental.pallas import tpu_sc as plsc

N = 10000
D_IN = 128
D_H1 = 128
D_H2 = 64

NUM_TILES = 32
SUBCORES = 16
BATCH = 128
ACC_ROWS = 10112
ROWS_PER_TILE = ACC_ROWS // SUBCORES
ROW_BLK = 1000

S_C0 = 40
S_C1 = 40


def _sc_degree(dst3, zeros_init, e0_rows):
  streams = dst3.shape[1]
  mesh = plsc.VectorSubcoreMesh(core_axis_name="c", subcore_axis_name="s")

  @functools.partial(
      pl.kernel,
      mesh=mesh,
      out_type=jax.ShapeDtypeStruct((2, ACC_ROWS, D_H1), jnp.float32),
      scratch_types=[
          pltpu.VMEM((streams, BATCH), jnp.int32),
          pltpu.VMEM((BATCH, D_H1), jnp.float32),
          pltpu.VMEM_SHARED((ACC_ROWS, D_H1), jnp.float32),
      ],
  )
  def k(dst_hbm, z_hbm, e0_hbm, out_hbm, dst_v, ones_v, acc):
    c = lax.axis_index("c")
    s = lax.axis_index("s")
    wid = s * 2 + c
    pltpu.sync_copy(z_hbm, acc.at[pl.ds(s * ROWS_PER_TILE, ROWS_PER_TILE)])
    pltpu.sync_copy(e0_hbm, ones_v)
    pltpu.sync_copy(dst_hbm.at[wid], dst_v)
    plsc.subcore_barrier()

    def body(j, carry):
      pltpu.sync_copy(ones_v, acc.at[dst_v.at[j]], add=True)
      return carry

    lax.fori_loop(0, streams, body, 0)
    plsc.subcore_barrier()
    pltpu.sync_copy(
        acc.at[pl.ds(s * ROWS_PER_TILE, ROWS_PER_TILE)],
        out_hbm.at[c, pl.ds(s * ROWS_PER_TILE, ROWS_PER_TILE)])

  return k(dst3, zeros_init, e0_rows)


def _sc_aggregate(y, src2, dst2, zeros_init):
  mesh = plsc.VectorSubcoreMesh(core_axis_name="c", subcore_axis_name="s")

  NBUF = 2
  SMAX = max(S_C0, S_C1)

  @functools.partial(
      pl.kernel,
      mesh=mesh,
      out_type=jax.ShapeDtypeStruct((2, ACC_ROWS, D_H1), jnp.float32),
      scratch_types=[
          pltpu.VMEM((SMAX, BATCH), jnp.int32),
          pltpu.VMEM((SMAX, BATCH), jnp.int32),
          [pltpu.VMEM((BATCH, D_H1), jnp.float32)] * NBUF,
          pltpu.VMEM_SHARED((ACC_ROWS, D_H1), jnp.float32),
          [pltpu.SemaphoreType.DMA] * NBUF,
          [pltpu.SemaphoreType.DMA] * NBUF,
      ],
  )
  def k(y_hbm, src_hbm, dst_hbm, z_hbm, out_hbm, src_v, dst_v, rows, acc,
        gsem, ssem):
    c = lax.axis_index("c")
    s = lax.axis_index("s")
    pltpu.sync_copy(z_hbm, acc.at[pl.ds(s * ROWS_PER_TILE, ROWS_PER_TILE)])
    plsc.subcore_barrier()

    def run(ns, base, yref):
      pltpu.sync_copy(src_hbm.at[pl.ds(base, ns)], src_v.at[pl.ds(0, ns)])
      pltpu.sync_copy(dst_hbm.at[pl.ds(base, ns)], dst_v.at[pl.ds(0, ns)])
      gd = [None] * NBUF
      sd = [None] * NBUF
      for b in range(min(NBUF, ns)):
        gd[b] = pltpu.async_copy(yref.at[src_v.at[b]], rows[b], gsem[b])
      for j in range(ns):
        p = j % NBUF
        gd[p].wait()
        sd[p] = pltpu.async_copy(rows[p], acc.at[dst_v.at[j]], ssem[p],
                                 add=True)
        nj = j + NBUF
        sd[p].wait()
        if nj < ns:
          gd[p] = pltpu.async_copy(yref.at[src_v.at[nj]], rows[p], gsem[p])

    @pl.when(c == 0)
    def _():
      run(S_C0, s * S_C0, y_hbm)

    @pl.when(c == 1)
    def _():
      run(S_C1, SUBCORES * S_C0 + s * S_C1, y_hbm)

    plsc.subcore_barrier()
    pltpu.sync_copy(
        acc.at[pl.ds(s * ROWS_PER_TILE, ROWS_PER_TILE)],
        out_hbm.at[c, pl.ds(s * ROWS_PER_TILE, ROWS_PER_TILE)])

  return k(y, src2, dst2, zeros_init)


def _tc_xw_scale(X, W1, degs):
  grid = N // ROW_BLK

  def body(x_ref, w_ref, deg_ref, y_ref, dinv_ref):
    deg = deg_ref[0, :, 0] + deg_ref[1, :, 0] + 1.0
    dinv = lax.rsqrt(jnp.maximum(deg, 1.0))
    xw = jnp.dot(x_ref[...], w_ref[...], preferred_element_type=jnp.float32)
    y_ref[...] = xw * dinv[:, None]
    dinv_ref[...] = dinv[:, None]

  return pl.pallas_call(
      body,
      grid=(grid,),
      in_specs=[
          pl.BlockSpec((ROW_BLK, D_IN), lambda i: (i, 0)),
          pl.BlockSpec((D_IN, D_H1), lambda i: (0, 0)),
          pl.BlockSpec((2, ROW_BLK, D_H1), lambda i: (0, i, 0)),
      ],
      out_specs=[
          pl.BlockSpec((ROW_BLK, D_H1), lambda i: (i, 0)),
          pl.BlockSpec((ROW_BLK, 1), lambda i: (i, 0)),
      ],
      out_shape=[
          jax.ShapeDtypeStruct((N, D_H1), jnp.float32),
          jax.ShapeDtypeStruct((N, 1), jnp.float32),
      ],
  )(X, W1, degs)


def _tc_combine(s_parts, y, dinv, b):
  grid = N // ROW_BLK

  def body(s_ref, y_ref, dinv_ref, b_ref, out_ref):
    di = dinv_ref[...]
    t = (s_ref[0] + s_ref[1] + y_ref[...]) * di + b_ref[...]
    out_ref[...] = jnp.maximum(t, 0.0) * di

  return pl.pallas_call(
      body,
      grid=(grid,),
      in_specs=[
          pl.BlockSpec((2, ROW_BLK, D_H1), lambda i: (0, i, 0)),
          pl.BlockSpec((ROW_BLK, D_H1), lambda i: (i, 0)),
          pl.BlockSpec((ROW_BLK, 1), lambda i: (i, 0)),
          pl.BlockSpec((1, D_H1), lambda i: (0, 0)),
      ],
      out_specs=pl.BlockSpec((ROW_BLK, D_H1), lambda i: (i, 0)),
      out_shape=jax.ShapeDtypeStruct((N, D_H1), jnp.float32),
  )(s_parts, y, dinv, b)


def _tc_heads(s_parts, y, dinv, Wm, bm, Wl, bl, noise):
  grid = N // ROW_BLK

  def body(s_ref, y_ref, dinv_ref, wm_ref, bm_ref, wl_ref, bl_ref, n_ref,
           z_ref):
    g = (s_ref[0] + s_ref[1] + y_ref[...]) * dinv_ref[...]
    mean = jnp.maximum(
        jnp.dot(g, wm_ref[...], preferred_element_type=jnp.float32)
        + bm_ref[...], 0.0)
    logstd = jnp.maximum(
        jnp.dot(g, wl_ref[...], preferred_element_type=jnp.float32)
        + bl_ref[...], 0.0)
    z_ref[...] = n_ref[...] * jnp.exp(logstd) + mean

  return pl.pallas_call(
      body,
      grid=(grid,),
      in_specs=[
          pl.BlockSpec((2, ROW_BLK, D_H1), lambda i: (0, i, 0)),
          pl.BlockSpec((ROW_BLK, D_H1), lambda i: (i, 0)),
          pl.BlockSpec((ROW_BLK, 1), lambda i: (i, 0)),
          pl.BlockSpec((D_H1, D_H2), lambda i: (0, 0)),
          pl.BlockSpec((1, D_H2), lambda i: (0, 0)),
          pl.BlockSpec((D_H1, D_H2), lambda i: (0, 0)),
          pl.BlockSpec((1, D_H2), lambda i: (0, 0)),
          pl.BlockSpec((ROW_BLK, D_H2), lambda i: (i, 0)),
      ],
      out_specs=pl.BlockSpec((ROW_BLK, D_H2), lambda i: (i, 0)),
      out_shape=jax.ShapeDtypeStruct((N, D_H2), jnp.float32),
  )(s_parts, y, dinv, Wm, bm, Wl, bl, noise)


def _tc_decode(Z):
  DEC_I = 2048
  DEC_J = 2048
  gi = pl.cdiv(N, DEC_I)
  gj = pl.cdiv(N, DEC_J)

  def body(a_ref, b_ref, out_ref):
    logits = lax.dot_general(
        a_ref[...], b_ref[...], (((1,), (1,)), ((), ())),
        preferred_element_type=jnp.float32)
    out_ref[...] = jax.nn.sigmoid(logits)

  return pl.pallas_call(
      body,
      grid=(gi, gj),
      in_specs=[
          pl.BlockSpec((DEC_I, D_H2), lambda i, j: (i, 0)),
          pl.BlockSpec((DEC_J, D_H2), lambda i, j: (j, 0)),
      ],
      out_specs=pl.BlockSpec((DEC_I, DEC_J), lambda i, j: (i, j)),
      out_shape=jax.ShapeDtypeStruct((N, N), jnp.float32),
  )(Z, Z)


def kernel(X, adj, W1, b1, Wm, bm, Wl, bl):
  E = adj.shape[1]
  src = adj[0].astype(jnp.int32)
  dst = adj[1].astype(jnp.int32)

  chunk = NUM_TILES * BATCH
  e_pad_deg = ((E + chunk - 1) // chunk) * chunk
  e_pad_agg = SUBCORES * (S_C0 + S_C1) * BATCH

  def padded(vec, n_to, dummy):
    return jnp.concatenate([vec, dummy[:n_to - E]])

  dummy_src = jnp.arange(max(e_pad_deg, e_pad_agg) - E, dtype=jnp.int32) % N
  dummy_dst = N + jnp.arange(max(e_pad_deg, e_pad_agg) - E,
                             dtype=jnp.int32) % (ACC_ROWS - N)

  streams = e_pad_deg // chunk
  dst3 = padded(dst, e_pad_deg, dummy_dst).reshape(NUM_TILES, streams, BATCH)
  src2 = padded(src, e_pad_agg, dummy_src).reshape(-1, BATCH)
  dst2 = padded(dst, e_pad_agg, dummy_dst).reshape(-1, BATCH)

  zeros_feat = jnp.zeros((ROWS_PER_TILE, D_H1), jnp.float32)
  zeros_deg = zeros_feat
  e0_rows = jnp.zeros((BATCH, D_H1), jnp.float32).at[:, 0].set(1.0)

  degs = _sc_degree(dst3, zeros_deg, e0_rows)
  y1, dinv = _tc_xw_scale(X, W1, degs)
  s1 = _sc_aggregate(y1, src2, dst2, zeros_feat)
  y2 = _tc_combine(s1, y1, dinv, b1.reshape(1, D_H1))
  s2 = _sc_aggregate(y2, src2, dst2, zeros_feat)
  noise = jax.random.normal(jax.random.key(42), (N, D_H2), dtype=jnp.float32)
  Z = _tc_heads(s2, y2, dinv, Wm, bm.reshape(1, D_H2), Wl,
                bl.reshape(1, D_H2), noise)
  A_pred = _tc_decode(Z)
  return (Z, A_pred)


# speedup vs baseline: 1.0730x; 1.0017x over previous
import functools

import jax
import jax.numpy as jnp
from jax import lax
from jax.experimental import pallas as pl
from jax.experimental.pallas import tpu as pltpu
from jax.experimental.pallas import tpu_sc as plsc

N = 10000
D_IN = 128
D_H1 = 128
D_H2 = 64

NUM_TILES = 32
SUBCORES = 16
BATCH = 128
ACC_ROWS = 10112
ROWS_PER_TILE = ACC_ROWS // SUBCORES
ROW_BLK = 1000

S_C0 = 40
S_C1 = 40


def _sc_degree(dst3, zeros_init, e0_rows):
  streams = dst3.shape[1]
  mesh = plsc.VectorSubcoreMesh(core_axis_name="c", subcore_axis_name="s")

  @functools.partial(
      pl.kernel,
      mesh=mesh,
      out_type=jax.ShapeDtypeStruct((2, ACC_ROWS, D_H1), jnp.float32),
      scratch_types=[
          pltpu.VMEM((streams, BATCH), jnp.int32),
          pltpu.VMEM((BATCH, D_H1), jnp.float32),
          pltpu.VMEM_SHARED((ACC_ROWS, D_H1), jnp.float32),
      ],
  )
  def k(dst_hbm, z_hbm, e0_hbm, out_hbm, dst_v, ones_v, acc):
    c = lax.axis_index("c")
    s = lax.axis_index("s")
    wid = s * 2 + c
    pltpu.sync_copy(z_hbm, acc.at[pl.ds(s * ROWS_PER_TILE, ROWS_PER_TILE)])
    pltpu.sync_copy(e0_hbm, ones_v)
    pltpu.sync_copy(dst_hbm.at[wid], dst_v)
    plsc.subcore_barrier()

    def body(j, carry):
      pltpu.sync_copy(ones_v, acc.at[dst_v.at[j]], add=True)
      return carry

    lax.fori_loop(0, streams, body, 0)
    plsc.subcore_barrier()
    pltpu.sync_copy(
        acc.at[pl.ds(s * ROWS_PER_TILE, ROWS_PER_TILE)],
        out_hbm.at[c, pl.ds(s * ROWS_PER_TILE, ROWS_PER_TILE)])

  return k(dst3, zeros_init, e0_rows)


def _sc_aggregate(y, src2, dst2, zeros_init):
  mesh = plsc.VectorSubcoreMesh(core_axis_name="c", subcore_axis_name="s")

  NBUF = 2
  SMAX = max(S_C0, S_C1)

  @functools.partial(
      pl.kernel,
      mesh=mesh,
      out_type=jax.ShapeDtypeStruct((2, ACC_ROWS, D_H1), jnp.float32),
      scratch_types=[
          pltpu.VMEM((SMAX, BATCH), jnp.int32),
          pltpu.VMEM((SMAX, BATCH), jnp.int32),
          [pltpu.VMEM((BATCH, D_H1), jnp.float32)] * NBUF,
          pltpu.VMEM_SHARED((ACC_ROWS, D_H1), jnp.float32),
          [pltpu.SemaphoreType.DMA] * NBUF,
          [pltpu.SemaphoreType.DMA] * NBUF,
      ],
  )
  def k(y_hbm, src_hbm, dst_hbm, z_hbm, out_hbm, src_v, dst_v, rows, acc,
        gsem, ssem):
    c = lax.axis_index("c")
    s = lax.axis_index("s")
    pltpu.sync_copy(z_hbm, acc.at[pl.ds(s * ROWS_PER_TILE, ROWS_PER_TILE)])
    plsc.subcore_barrier()

    def run(ns, base, yref):
      pltpu.sync_copy(src_hbm.at[pl.ds(base, ns)], src_v.at[pl.ds(0, ns)])
      pltpu.sync_copy(dst_hbm.at[pl.ds(base, ns)], dst_v.at[pl.ds(0, ns)])
      gd = [None] * NBUF
      sd = [None] * NBUF
      for b in range(min(NBUF, ns)):
        gd[b] = pltpu.async_copy(yref.at[src_v.at[b]], rows[b], gsem[b])
      for j in range(ns):
        p = j % NBUF
        gd[p].wait()
        sd[p] = pltpu.async_copy(rows[p], acc.at[dst_v.at[j]], ssem[p],
                                 add=True)
        nj = j + NBUF
        sd[p].wait()
        if nj < ns:
          gd[p] = pltpu.async_copy(yref.at[src_v.at[nj]], rows[p], gsem[p])

    @pl.when(c == 0)
    def _():
      run(S_C0, s * S_C0, y_hbm)

    @pl.when(c == 1)
    def _():
      run(S_C1, SUBCORES * S_C0 + s * S_C1, y_hbm)

    plsc.subcore_barrier()
    pltpu.sync_copy(
        acc.at[pl.ds(s * ROWS_PER_TILE, ROWS_PER_TILE)],
        out_hbm.at[c, pl.ds(s * ROWS_PER_TILE, ROWS_PER_TILE)])

  return k(y, src2, dst2, zeros_init)


def _tc_xw_scale(X, W1, degs):
  grid = N // ROW_BLK

  def body(x_ref, w_ref, deg_ref, y_ref, dinv_ref):
    deg = deg_ref[0, :, 0] + deg_ref[1, :, 0] + 1.0
    dinv = lax.rsqrt(jnp.maximum(deg, 1.0))
    xw = jnp.dot(x_ref[...], w_ref[...], preferred_element_type=jnp.float32)
    y_ref[...] = xw * dinv[:, None]
    dinv_ref[...] = dinv[:, None]

  return pl.pallas_call(
      body,
      grid=(grid,),
      in_specs=[
          pl.BlockSpec((ROW_BLK, D_IN), lambda i: (i, 0)),
          pl.BlockSpec((D_IN, D_H1), lambda i: (0, 0)),
          pl.BlockSpec((2, ROW_BLK, D_H1), lambda i: (0, i, 0)),
      ],
      out_specs=[
          pl.BlockSpec((ROW_BLK, D_H1), lambda i: (i, 0)),
          pl.BlockSpec((ROW_BLK, 1), lambda i: (i, 0)),
      ],
      out_shape=[
          jax.ShapeDtypeStruct((N, D_H1), jnp.float32),
          jax.ShapeDtypeStruct((N, 1), jnp.float32),
      ],
  )(X, W1, degs)


def _tc_combine(s_parts, y, dinv, b):
  grid = N // ROW_BLK

  def body(s_ref, y_ref, dinv_ref, b_ref, out_ref):
    di = dinv_ref[...]
    t = (s_ref[0] + s_ref[1] + y_ref[...]) * di + b_ref[...]
    out_ref[...] = jnp.maximum(t, 0.0) * di

  return pl.pallas_call(
      body,
      grid=(grid,),
      in_specs=[
          pl.BlockSpec((2, ROW_BLK, D_H1), lambda i: (0, i, 0)),
          pl.BlockSpec((ROW_BLK, D_H1), lambda i: (i, 0)),
          pl.BlockSpec((ROW_BLK, 1), lambda i: (i, 0)),
          pl.BlockSpec((1, D_H1), lambda i: (0, 0)),
      ],
      out_specs=pl.BlockSpec((ROW_BLK, D_H1), lambda i: (i, 0)),
      out_shape=jax.ShapeDtypeStruct((N, D_H1), jnp.float32),
  )(s_parts, y, dinv, b)


def _tc_heads(s_parts, y, dinv, Wm, bm, Wl, bl, noise):
  grid = N // ROW_BLK

  def body(s_ref, y_ref, dinv_ref, wm_ref, bm_ref, wl_ref, bl_ref, n_ref,
           z_ref):
    g = (s_ref[0] + s_ref[1] + y_ref[...]) * dinv_ref[...]
    mean = jnp.maximum(
        jnp.dot(g, wm_ref[...], preferred_element_type=jnp.float32)
        + bm_ref[...], 0.0)
    logstd = jnp.maximum(
        jnp.dot(g, wl_ref[...], preferred_element_type=jnp.float32)
        + bl_ref[...], 0.0)
    z_ref[...] = n_ref[...] * jnp.exp(logstd) + mean

  return pl.pallas_call(
      body,
      grid=(grid,),
      in_specs=[
          pl.BlockSpec((2, ROW_BLK, D_H1), lambda i: (0, i, 0)),
          pl.BlockSpec((ROW_BLK, D_H1), lambda i: (i, 0)),
          pl.BlockSpec((ROW_BLK, 1), lambda i: (i, 0)),
          pl.BlockSpec((D_H1, D_H2), lambda i: (0, 0)),
          pl.BlockSpec((1, D_H2), lambda i: (0, 0)),
          pl.BlockSpec((D_H1, D_H2), lambda i: (0, 0)),
          pl.BlockSpec((1, D_H2), lambda i: (0, 0)),
          pl.BlockSpec((ROW_BLK, D_H2), lambda i: (i, 0)),
      ],
      out_specs=pl.BlockSpec((ROW_BLK, D_H2), lambda i: (i, 0)),
      out_shape=jax.ShapeDtypeStruct((N, D_H2), jnp.float32),
  )(s_parts, y, dinv, Wm, bm, Wl, bl, noise)


def _tc_decode(Z):
  DEC_I = 2560
  DEC_J = 2048
  gi = pl.cdiv(N, DEC_I)
  gj = pl.cdiv(N, DEC_J)

  def body(a_ref, b_ref, out_ref):
    logits = lax.dot_general(
        a_ref[...], b_ref[...], (((1,), (1,)), ((), ())),
        preferred_element_type=jnp.float32)
    out_ref[...] = jax.nn.sigmoid(logits)

  return pl.pallas_call(
      body,
      grid=(gi, gj),
      in_specs=[
          pl.BlockSpec((DEC_I, D_H2), lambda i, j: (i, 0)),
          pl.BlockSpec((DEC_J, D_H2), lambda i, j: (j, 0)),
      ],
      out_specs=pl.BlockSpec((DEC_I, DEC_J), lambda i, j: (i, j)),
      out_shape=jax.ShapeDtypeStruct((N, N), jnp.float32),
  )(Z, Z)


def kernel(X, adj, W1, b1, Wm, bm, Wl, bl):
  E = adj.shape[1]
  src = adj[0].astype(jnp.int32)
  dst = adj[1].astype(jnp.int32)

  chunk = NUM_TILES * BATCH
  e_pad_deg = ((E + chunk - 1) // chunk) * chunk
  e_pad_agg = SUBCORES * (S_C0 + S_C1) * BATCH

  def padded(vec, n_to, dummy):
    return jnp.concatenate([vec, dummy[:n_to - E]])

  dummy_src = jnp.arange(max(e_pad_deg, e_pad_agg) - E, dtype=jnp.int32) % N
  dummy_dst = N + jnp.arange(max(e_pad_deg, e_pad_agg) - E,
                             dtype=jnp.int32) % (ACC_ROWS - N)

  streams = e_pad_deg // chunk
  dst3 = padded(dst, e_pad_deg, dummy_dst).reshape(NUM_TILES, streams, BATCH)
  src2 = padded(src, e_pad_agg, dummy_src).reshape(-1, BATCH)
  dst2 = padded(dst, e_pad_agg, dummy_dst).reshape(-1, BATCH)

  zeros_feat = jnp.zeros((ROWS_PER_TILE, D_H1), jnp.float32)
  zeros_deg = zeros_feat
  e0_rows = jnp.zeros((BATCH, D_H1), jnp.float32).at[:, 0].set(1.0)

  degs = _sc_degree(dst3, zeros_deg, e0_rows)
  y1, dinv = _tc_xw_scale(X, W1, degs)
  s1 = _sc_aggregate(y1, src2, dst2, zeros_feat)
  y2 = _tc_combine(s1, y1, dinv, b1.reshape(1, D_H1))
  s2 = _sc_aggregate(y2, src2, dst2, zeros_feat)
  noise = jax.random.normal(jax.random.key(42), (N, D_H2), dtype=jnp.float32)
  Z = _tc_heads(s2, y2, dinv, Wm, bm.reshape(1, D_H2), Wl,
                bl.reshape(1, D_H2), noise)
  A_pred = _tc_decode(Z)
  return (Z, A_pred)
